# baseline (speedup 1.0000x reference)
import functools

import jax
import jax.numpy as jnp
from jax import lax
from jax.experimental import pallas as pl
from jax.experimental.pallas import tpu as pltpu
from jax.experimental.pallas import tpu_sc as plsc

NUM_TOKEN_FIELDS = 26
VOCAB = 100000
D = 16
BATCH = 4096
HIST = 50
NUM_FIELDS = NUM_TOKEN_FIELDS + 2

NC = 2
NS = 16
NW = NC * NS
BW = BATCH // NW
SEQ_CHUNK = 64
N_CHUNKS = BW // SEQ_CHUNK
SEQ_STREAMS = SEQ_CHUNK * HIST // 128
TOK_HALF = NUM_TOKEN_FIELDS // 2


def _sc_body(tok_tab, seq_tab, tok_idx, seq_idx, ff, dw, gvec, bvec,
             out,
             tidx_raw, tidx_v, tok_v, sidx_v, seq_rows,
             pooled_v, cntinv_v, ff_v, dense_v, dw_v, g_v, b_v,
             sem_g, sem_w):
    wid = lax.axis_index("s") * NC + lax.axis_index("c")
    base = wid * BW
    iota = lax.iota(jnp.int32, 16)
    zeros16 = jnp.zeros((16,), jnp.float32)

    pltpu.sync_copy(tok_idx.at[pl.ds(base * NUM_TOKEN_FIELDS,
                                     BW * NUM_TOKEN_FIELDS)], tidx_raw)
    pltpu.sync_copy(seq_idx.at[pl.ds(base * HIST, BW * HIST)], sidx_v)
    pltpu.sync_copy(ff, ff_v)
    pltpu.sync_copy(dw, dw_v)
    pltpu.sync_copy(gvec, g_v)
    pltpu.sync_copy(bvec, b_v)

    ngrp = BW // 16

    def tok_xform(j, carry):
        f = j // ngrp
        c = j - f * ngrp
        pos = (iota + c * 16) * NUM_TOKEN_FIELDS + f
        v = plsc.load_gather(tidx_raw, [pos])
        tidx_v[pl.ds(f * BW + c * 16, 16)] = v + f * VOCAB
        return carry

    lax.fori_loop(0, NUM_TOKEN_FIELDS * ngrp, tok_xform, 0)

    tok_descs = [
        pltpu.async_copy(tok_tab.at[tidx_v.at[pl.ds(f * BW, BW)]],
                         tok_v.at[pl.ds(f * BW, BW)], sem_g)
        for f in range(NUM_TOKEN_FIELDS)
    ]

    def cnt_group(g, carry):
        rows = (iota + g * 16) * HIST

        def cnt_l(l, acc):
            v = plsc.load_gather(sidx_v, [rows + l])
            return acc + jnp.where(v > 0, 1.0, 0.0).astype(jnp.float32)

        cnt = lax.fori_loop(0, HIST, cnt_l, zeros16)
        cntinv_v[pl.ds(g * 16, 16)] = 1.0 / jnp.maximum(cnt, 1.0)
        return carry

    lax.fori_loop(0, BW // 16, cnt_group, 0)

    def stat_body(i, carry):
        s, s2 = carry
        v = ff_v[pl.ds(i * 16, 16)]
        return (s + v, s2 + v * v)

    s, s2 = lax.fori_loop(0, BATCH // 16, stat_body, (zeros16, zeros16))
    mean = jnp.sum(s) * (1.0 / BATCH)
    var = jnp.sum(s2) * (1.0 / BATCH) - mean * mean
    x = jnp.full((16,), var + 1e-5, jnp.float32)
    y = plsc.bitcast(jnp.int32(0x5F3759DF)
                     - lax.shift_right_arithmetic(plsc.bitcast(x, jnp.int32), 1),
                     jnp.float32)
    for _ in range(3):
        y = y * (1.5 - 0.5 * x * y * y)
    gamma = g_v[...]
    beta = b_v[...]
    scale = y * gamma
    mean_v = jnp.full((16,), mean, jnp.float32)
    dw_vec = dw_v[...]

    def dense_body(b, carry):
        xv = plsc.load_gather(ff_v, [jnp.full((16,), base + b, jnp.int32)])
        emb = ((xv - mean_v) * scale + beta) * dw_vec
        dense_v[b] = emb
        return carry

    lax.fori_loop(0, BW, dense_body, 0)

    for d_ in tok_descs:
        d_.wait()
    w_descs = [
        pltpu.async_copy(tok_v.at[pl.ds(f * BW, BW)],
                         out.at[pl.ds(base, BW), f], sem_w)
        for f in range(NUM_TOKEN_FIELDS)
    ]

    for cc in range(N_CHUNKS):
        s_descs = [
            pltpu.async_copy(
                seq_tab.at[sidx_v.at[pl.ds(cc * SEQ_CHUNK * HIST + j * 128,
                                           128)]],
                seq_rows.at[pl.ds(j * 128, 128)], sem_g)
            for j in range(SEQ_STREAMS)
        ]
        for d_ in s_descs:
            d_.wait()

        def pool_body(bb, carry):
            r0 = bb * HIST
            a0 = zeros16
            a1 = zeros16
            for l in range(0, HIST, 2):
                a0 = a0 + seq_rows[r0 + l]
                a1 = a1 + seq_rows[r0 + l + 1]
            b = cc * SEQ_CHUNK + bb
            inv = plsc.load_gather(cntinv_v, [jnp.full((16,), b, jnp.int32)])
            pooled_v[b] = (a0 + a1) * inv
            return carry

        lax.fori_loop(0, SEQ_CHUNK, pool_body, 0)

    pw = pltpu.async_copy(pooled_v, out.at[pl.ds(base, BW), NUM_TOKEN_FIELDS],
                          sem_w)
    dwr = pltpu.async_copy(dense_v,
                           out.at[pl.ds(base, BW), NUM_TOKEN_FIELDS + 1],
                           sem_w)
    for d_ in w_descs:
        d_.wait()
    pw.wait()
    dwr.wait()


def _build_call(interpret=False):
    return functools.partial(
        pl.kernel,
        out_type=jax.ShapeDtypeStruct((BATCH, NUM_FIELDS, D), jnp.float32),
        mesh=plsc.VectorSubcoreMesh(core_axis_name="c", subcore_axis_name="s",
                                    num_cores=NC, num_subcores=NS),
        compiler_params=pltpu.CompilerParams(needs_layout_passes=False,
                                             use_tc_tiling_on_sc=False),
        interpret=interpret,
        scratch_types=[
        pltpu.VMEM((BW * NUM_TOKEN_FIELDS,), jnp.int32),
        pltpu.VMEM((NUM_TOKEN_FIELDS * BW,), jnp.int32),
        pltpu.VMEM((NUM_TOKEN_FIELDS * BW, D), jnp.float32),
        pltpu.VMEM((BW * HIST,), jnp.int32),
        pltpu.VMEM((SEQ_CHUNK * HIST, D), jnp.float32),
        pltpu.VMEM((BW, D), jnp.float32),
        pltpu.VMEM((BW,), jnp.float32),
        pltpu.VMEM((BATCH,), jnp.float32),
        pltpu.VMEM((BW, D), jnp.float32),
        pltpu.VMEM((D,), jnp.float32),
        pltpu.VMEM((16,), jnp.float32),
        pltpu.VMEM((16,), jnp.float32),
        pltpu.SemaphoreType.DMA,
        pltpu.SemaphoreType.DMA,
        ],
    )(_sc_body)


_sc_call = _build_call()


def kernel(token_idx, seq_idx, float_feat, token_tables, seq_table,
           dense_w, bn_gamma, bn_beta):
    tok_idx = token_idx.astype(jnp.int32).reshape(-1)
    s_idx = seq_idx.astype(jnp.int32).reshape(-1)
    one = bn_gamma.astype(jnp.float32) * 0.0 + 1.0
    tok_tab = (token_tables * one).reshape(NUM_TOKEN_FIELDS * VOCAB, D)
    gvec = jnp.full((16,), bn_gamma, jnp.float32)
    bvec = jnp.full((16,), bn_beta, jnp.float32)
    return _sc_call(tok_tab, seq_table.astype(jnp.float32) * one, tok_idx, s_idx,
                    float_feat.astype(jnp.float32),
                    dense_w.astype(jnp.float32), gvec, bvec)


# speedup vs baseline: 1.1095x; 1.1095x over previous
import functools

import jax
import jax.numpy as jnp
from jax import lax
from jax.experimental import pallas as pl
from jax.experimental.pallas import tpu as pltpu
from jax.experimental.pallas import tpu_sc as plsc

NUM_TOKEN_FIELDS = 26
VOCAB = 100000
D = 16
BATCH = 4096
HIST = 50
NUM_FIELDS = NUM_TOKEN_FIELDS + 2

NC = 2
NS = 16
NW = NC * NS
BW = BATCH // NW
SEQ_CHUNK = 64
N_CHUNKS = BW // SEQ_CHUNK
SEQ_STREAMS = SEQ_CHUNK * HIST // 128
TOK_HALF = NUM_TOKEN_FIELDS // 2


def _sc_body(tok_tab, seq_tab, tok_idx, seq_idx, ff, dw, gvec, bvec,
             out,
             tidx_raw, tidx_v, tok_v, sidx_v, seq_rows,
             pooled_v, cntinv_v, ff_v, dense_v, dw_v, g_v, b_v,
             sem_g, sem_w):
    wid = lax.axis_index("s") * NC + lax.axis_index("c")
    base = wid * BW
    iota = lax.iota(jnp.int32, 16)
    zeros16 = jnp.zeros((16,), jnp.float32)

    pltpu.sync_copy(tok_idx.at[pl.ds(base * NUM_TOKEN_FIELDS,
                                     BW * NUM_TOKEN_FIELDS)], tidx_raw)
    pltpu.sync_copy(seq_idx.at[pl.ds(base * HIST, BW * HIST)], sidx_v)
    pltpu.sync_copy(ff, ff_v)
    pltpu.sync_copy(dw, dw_v)
    pltpu.sync_copy(gvec, g_v)
    pltpu.sync_copy(bvec, b_v)

    ngrp = BW // 16

    def tok_xform(j, carry):
        f = j // ngrp
        c = j - f * ngrp
        pos = (iota + c * 16) * NUM_TOKEN_FIELDS + f
        v = plsc.load_gather(tidx_raw, [pos])
        tidx_v[pl.ds(f * BW + c * 16, 16)] = v + f * VOCAB
        return carry

    lax.fori_loop(0, NUM_TOKEN_FIELDS * ngrp, tok_xform, 0)

    tok_descs = [
        pltpu.async_copy(tok_tab.at[tidx_v.at[pl.ds(f * BW, BW)]],
                         tok_v.at[pl.ds(f * BW, BW)], sem_g)
        for f in range(NUM_TOKEN_FIELDS)
    ]

    def cnt_group(g, carry):
        rows = (iota + g * 16) * HIST

        def cnt_l(l, acc):
            v = plsc.load_gather(sidx_v, [rows + l])
            return acc + jnp.where(v > 0, 1.0, 0.0).astype(jnp.float32)

        cnt = lax.fori_loop(0, HIST, cnt_l, zeros16)
        cntinv_v[pl.ds(g * 16, 16)] = 1.0 / jnp.maximum(cnt, 1.0)
        return carry

    lax.fori_loop(0, BW // 16, cnt_group, 0)

    def stat_body(i, carry):
        s, s2 = carry
        v = ff_v[pl.ds(i * 16, 16)]
        return (s + v, s2 + v * v)

    s, s2 = lax.fori_loop(0, BATCH // 16, stat_body, (zeros16, zeros16))
    mean = jnp.sum(s) * (1.0 / BATCH)
    var = jnp.sum(s2) * (1.0 / BATCH) - mean * mean
    x = jnp.full((16,), var + 1e-5, jnp.float32)
    y = plsc.bitcast(jnp.int32(0x5F3759DF)
                     - lax.shift_right_arithmetic(plsc.bitcast(x, jnp.int32), 1),
                     jnp.float32)
    for _ in range(3):
        y = y * (1.5 - 0.5 * x * y * y)
    gamma = g_v[...]
    beta = b_v[...]
    scale = y * gamma
    mean_v = jnp.full((16,), mean, jnp.float32)
    dw_vec = dw_v[...]

    def dense_body(b, carry):
        xv = plsc.load_gather(ff_v, [jnp.full((16,), base + b, jnp.int32)])
        emb = ((xv - mean_v) * scale + beta) * dw_vec
        dense_v[b] = emb
        return carry

    lax.fori_loop(0, BW, dense_body, 0)

    for d_ in tok_descs:
        d_.wait()
    w_descs = [
        pltpu.async_copy(tok_v.at[pl.ds(f * BW, BW)],
                         out.at[pl.ds(base, BW), f], sem_w)
        for f in range(NUM_TOKEN_FIELDS)
    ]

    for cc in range(N_CHUNKS):
        s_descs = [
            pltpu.async_copy(
                seq_tab.at[sidx_v.at[pl.ds(cc * SEQ_CHUNK * HIST + j * 128,
                                           128)]],
                seq_rows.at[pl.ds(j * 128, 128)], sem_g)
            for j in range(SEQ_STREAMS)
        ]
        for d_ in s_descs:
            d_.wait()

        def pool_body(bb, carry):
            r0 = bb * HIST
            a0 = zeros16
            a1 = zeros16
            for l in range(0, HIST, 2):
                a0 = a0 + seq_rows[r0 + l]
                a1 = a1 + seq_rows[r0 + l + 1]
            b = cc * SEQ_CHUNK + bb
            inv = plsc.load_gather(cntinv_v, [jnp.full((16,), b, jnp.int32)])
            pooled_v[b] = (a0 + a1) * inv
            return carry

        lax.fori_loop(0, SEQ_CHUNK, pool_body, 0)

    pw = pltpu.async_copy(pooled_v, out.at[pl.ds(base, BW), NUM_TOKEN_FIELDS],
                          sem_w)
    dwr = pltpu.async_copy(dense_v,
                           out.at[pl.ds(base, BW), NUM_TOKEN_FIELDS + 1],
                           sem_w)
    for d_ in w_descs:
        d_.wait()
    pw.wait()
    dwr.wait()


def _build_call(interpret=False):
    return functools.partial(
        pl.kernel,
        out_type=jax.ShapeDtypeStruct((BATCH, NUM_FIELDS, D), jnp.float32),
        mesh=plsc.VectorSubcoreMesh(core_axis_name="c", subcore_axis_name="s",
                                    num_cores=NC, num_subcores=NS),
        compiler_params=pltpu.CompilerParams(needs_layout_passes=False,
                                             use_tc_tiling_on_sc=False),
        interpret=interpret,
        scratch_types=[
        pltpu.VMEM((BW * NUM_TOKEN_FIELDS,), jnp.int32),
        pltpu.VMEM((NUM_TOKEN_FIELDS * BW,), jnp.int32),
        pltpu.VMEM((NUM_TOKEN_FIELDS * BW, D), jnp.float32),
        pltpu.VMEM((BW * HIST,), jnp.int32),
        pltpu.VMEM((SEQ_CHUNK * HIST, D), jnp.float32),
        pltpu.VMEM((BW, D), jnp.float32),
        pltpu.VMEM((BW,), jnp.float32),
        pltpu.VMEM((BATCH,), jnp.float32),
        pltpu.VMEM((BW, D), jnp.float32),
        pltpu.VMEM((D,), jnp.float32),
        pltpu.VMEM((16,), jnp.float32),
        pltpu.VMEM((16,), jnp.float32),
        pltpu.SemaphoreType.DMA,
        pltpu.SemaphoreType.DMA,
        ],
    )(_sc_body)


_sc_call = _build_call()


def _retile_body(src, out, buf_v, tile_v, sem):
    wid = lax.axis_index("s") * NC + lax.axis_index("c")
    base = wid * BW
    iota = lax.iota(jnp.int32, 16)
    pltpu.sync_copy(src.at[pl.ds(base * NUM_FIELDS * D, BW * NUM_FIELDS * D)],
                    buf_v)

    def fld_body(fld, carry):
        def d_body(dd, carry2):
            def c_body(c, carry3):
                pos = (iota + c * 16) * (NUM_FIELDS * D) + fld * D + dd
                v = plsc.load_gather(buf_v, [pos])
                tile_v[dd, pl.ds(c * 16, 16)] = v
                return carry3
            lax.fori_loop(0, BW // 16, c_body, 0)
            return carry2
        lax.fori_loop(0, D, d_body, 0)
        pltpu.sync_copy(tile_v, out.at[fld, pl.ds(0, D), pl.ds(base, BW)])
        return carry

    lax.fori_loop(0, NUM_FIELDS, fld_body, 0)


_retile_call = functools.partial(
    pl.kernel,
    out_type=jax.ShapeDtypeStruct((NUM_FIELDS, D, BATCH), jnp.float32),
    mesh=plsc.VectorSubcoreMesh(core_axis_name="c", subcore_axis_name="s",
                                num_cores=NC, num_subcores=NS),
    compiler_params=pltpu.CompilerParams(needs_layout_passes=False,
                                         use_tc_tiling_on_sc=True),
    scratch_types=[
        pltpu.VMEM((BW * NUM_FIELDS * D,), jnp.float32),
        pltpu.VMEM((D, BW), jnp.float32),
        pltpu.SemaphoreType.DMA,
    ],
)(_retile_body)


def kernel(token_idx, seq_idx, float_feat, token_tables, seq_table,
           dense_w, bn_gamma, bn_beta):
    tok_idx = token_idx.astype(jnp.int32).reshape(-1)
    s_idx = seq_idx.astype(jnp.int32).reshape(-1)
    tok_tab = token_tables.reshape(NUM_TOKEN_FIELDS * VOCAB, D)
    gvec = jnp.full((16,), bn_gamma, jnp.float32)
    bvec = jnp.full((16,), bn_beta, jnp.float32)
    out_lin = _sc_call(tok_tab, seq_table.astype(jnp.float32), tok_idx, s_idx,
                       float_feat.astype(jnp.float32),
                       dense_w.astype(jnp.float32), gvec, bvec)
    out3 = _retile_call(out_lin.reshape(-1))
    return jnp.transpose(out3, (2, 0, 1))


# speedup vs baseline: 1.1114x; 1.0017x over previous
import functools

import jax
import jax.numpy as jnp
from jax import lax
from jax.experimental import pallas as pl
from jax.experimental.pallas import tpu as pltpu
from jax.experimental.pallas import tpu_sc as plsc

NUM_TOKEN_FIELDS = 26
VOCAB = 100000
D = 16
BATCH = 4096
HIST = 50
NUM_FIELDS = NUM_TOKEN_FIELDS + 2

NC = 2
NS = 16
NW = NC * NS
BW = BATCH // NW
SEQ_CHUNK = 64
N_CHUNKS = BW // SEQ_CHUNK
SEQ_STREAMS = SEQ_CHUNK * HIST // 128


def _sc_body(tok_tab, seq_tab, tok_idx, seq_idx, ff, dw, gvec, bvec,
             out,
             tidx_raw, tidx_v, tok_v, sidx_v, seq_rows,
             pooled_v, cntinv_v, ff_v, dense_v, dw_v, g_v, b_v,
             sem_g, sem_w):
    wid = lax.axis_index("s") * NC + lax.axis_index("c")
    base = wid * BW
    iota = lax.iota(jnp.int32, 16)
    zeros16 = jnp.zeros((16,), jnp.float32)

    pltpu.sync_copy(tok_idx.at[pl.ds(base * NUM_TOKEN_FIELDS,
                                     BW * NUM_TOKEN_FIELDS)], tidx_raw)
    pltpu.sync_copy(seq_idx.at[pl.ds(base * HIST, BW * HIST)], sidx_v)
    pltpu.sync_copy(ff, ff_v)
    pltpu.sync_copy(dw, dw_v)
    pltpu.sync_copy(gvec, g_v)
    pltpu.sync_copy(bvec, b_v)

    ngrp = BW // 16

    def tok_xform(j, carry):
        f = j // ngrp
        c = j - f * ngrp
        pos = (iota + c * 16) * NUM_TOKEN_FIELDS + f
        v = plsc.load_gather(tidx_raw, [pos])
        tidx_v[pl.ds(f * BW + c * 16, 16)] = v + f * VOCAB
        return carry

    lax.fori_loop(0, NUM_TOKEN_FIELDS * ngrp, tok_xform, 0)

    tok_descs = [
        pltpu.async_copy(tok_tab.at[tidx_v.at[pl.ds(f * BW, BW)]],
                         tok_v.at[pl.ds(f * BW, BW)], sem_g)
        for f in range(NUM_TOKEN_FIELDS)
    ]

    def cnt_group(g, carry):
        rows = (iota + g * 16) * HIST

        def cnt_l(l, acc):
            v = plsc.load_gather(sidx_v, [rows + l])
            return acc + jnp.where(v > 0, 1.0, 0.0).astype(jnp.float32)

        cnt = lax.fori_loop(0, HIST, cnt_l, zeros16)
        cntinv_v[pl.ds(g * 16, 16)] = 1.0 / jnp.maximum(cnt, 1.0)
        return carry

    lax.fori_loop(0, BW // 16, cnt_group, 0)

    def stat_body(i, carry):
        s, s2 = carry
        v = ff_v[pl.ds(i * 16, 16)]
        return (s + v, s2 + v * v)

    s, s2 = lax.fori_loop(0, BATCH // 16, stat_body, (zeros16, zeros16))
    mean = jnp.sum(s) * (1.0 / BATCH)
    var = jnp.sum(s2) * (1.0 / BATCH) - mean * mean
    x = jnp.full((16,), var + 1e-5, jnp.float32)
    y = plsc.bitcast(jnp.int32(0x5F3759DF)
                     - lax.shift_right_arithmetic(plsc.bitcast(x, jnp.int32), 1),
                     jnp.float32)
    for _ in range(3):
        y = y * (1.5 - 0.5 * x * y * y)
    gamma = g_v[...]
    beta = b_v[...]
    scale = y * gamma
    mean_v = jnp.full((16,), mean, jnp.float32)
    dw_vec = dw_v[...]

    def dense_body(b, carry):
        xv = plsc.load_gather(ff_v, [jnp.full((16,), base + b, jnp.int32)])
        emb = ((xv - mean_v) * scale + beta) * dw_vec
        dense_v[b] = emb
        return carry

    lax.fori_loop(0, BW, dense_body, 0)

    for d_ in tok_descs:
        d_.wait()
    w_descs = [
        pltpu.async_copy(tok_v.at[pl.ds(f * BW, BW)],
                         out.at[pl.ds(base, BW), f], sem_w)
        for f in range(NUM_TOKEN_FIELDS)
    ]

    for cc in range(N_CHUNKS):
        s_descs = [
            pltpu.async_copy(
                seq_tab.at[sidx_v.at[pl.ds(cc * SEQ_CHUNK * HIST + j * 128,
                                           128)]],
                seq_rows.at[pl.ds(j * 128, 128)], sem_g)
            for j in range(SEQ_STREAMS)
        ]
        for d_ in s_descs:
            d_.wait()

        def pool_body(bb, carry):
            r0 = bb * HIST
            a0 = zeros16
            a1 = zeros16
            for l in range(0, HIST, 2):
                a0 = a0 + seq_rows[r0 + l]
                a1 = a1 + seq_rows[r0 + l + 1]
            b = cc * SEQ_CHUNK + bb
            inv = plsc.load_gather(cntinv_v, [jnp.full((16,), b, jnp.int32)])
            pooled_v[b] = (a0 + a1) * inv
            return carry

        lax.fori_loop(0, SEQ_CHUNK, pool_body, 0)

    pw = pltpu.async_copy(pooled_v, out.at[pl.ds(base, BW), NUM_TOKEN_FIELDS],
                          sem_w)
    dwr = pltpu.async_copy(dense_v,
                           out.at[pl.ds(base, BW), NUM_TOKEN_FIELDS + 1],
                           sem_w)
    for d_ in w_descs:
        d_.wait()
    pw.wait()
    dwr.wait()


def _build_call(interpret=False):
    return functools.partial(
        pl.kernel,
        out_type=jax.ShapeDtypeStruct((BATCH, NUM_FIELDS, D), jnp.float32),
        mesh=plsc.VectorSubcoreMesh(core_axis_name="c", subcore_axis_name="s",
                                    num_cores=NC, num_subcores=NS),
        compiler_params=pltpu.CompilerParams(needs_layout_passes=False,
                                             use_tc_tiling_on_sc=False),
        interpret=interpret,
        scratch_types=[
        pltpu.VMEM((BW * NUM_TOKEN_FIELDS,), jnp.int32),
        pltpu.VMEM((NUM_TOKEN_FIELDS * BW,), jnp.int32),
        pltpu.VMEM((NUM_TOKEN_FIELDS * BW, D), jnp.float32),
        pltpu.VMEM((BW * HIST,), jnp.int32),
        pltpu.VMEM((SEQ_CHUNK * HIST, D), jnp.float32),
        pltpu.VMEM((BW, D), jnp.float32),
        pltpu.VMEM((BW,), jnp.float32),
        pltpu.VMEM((BATCH,), jnp.float32),
        pltpu.VMEM((BW, D), jnp.float32),
        pltpu.VMEM((D,), jnp.float32),
        pltpu.VMEM((16,), jnp.float32),
        pltpu.VMEM((16,), jnp.float32),
        pltpu.SemaphoreType.DMA,
        pltpu.SemaphoreType.DMA,
        ],
    )(_sc_body)


_sc_call = _build_call()


def _retile_body(src, out, buf_v, tile_v, sem):
    wid = lax.axis_index("s") * NC + lax.axis_index("c")
    base = wid * BW
    iota = lax.iota(jnp.int32, 16)
    pltpu.sync_copy(src.at[pl.ds(base * NUM_FIELDS * D, BW * NUM_FIELDS * D)],
                    buf_v)

    row = iota * (NUM_FIELDS * D)

    def fld_body(fld, carry):
        def d_body(dd, carry2):
            p0 = row + (fld * D + dd)
            for c in range(BW // 16):
                v = plsc.load_gather(buf_v, [p0 + c * 16 * (NUM_FIELDS * D)])
                tile_v[dd, pl.ds(c * 16, 16)] = v
            return carry2
        lax.fori_loop(0, D, d_body, 0)
        pltpu.sync_copy(tile_v, out.at[fld, pl.ds(0, D), pl.ds(base, BW)])
        return carry

    lax.fori_loop(0, NUM_FIELDS, fld_body, 0)


_retile_call = functools.partial(
    pl.kernel,
    out_type=jax.ShapeDtypeStruct((NUM_FIELDS, D, BATCH), jnp.float32),
    mesh=plsc.VectorSubcoreMesh(core_axis_name="c", subcore_axis_name="s",
                                num_cores=NC, num_subcores=NS),
    compiler_params=pltpu.CompilerParams(needs_layout_passes=False,
                                         use_tc_tiling_on_sc=True),
    scratch_types=[
        pltpu.VMEM((BW * NUM_FIELDS * D,), jnp.float32),
        pltpu.VMEM((D, BW), jnp.float32),
        pltpu.SemaphoreType.DMA,
    ],
)(_retile_body)


def kernel(token_idx, seq_idx, float_feat, token_tables, seq_table,
           dense_w, bn_gamma, bn_beta):
    tok_idx = token_idx.astype(jnp.int32).reshape(-1)
    s_idx = seq_idx.astype(jnp.int32).reshape(-1)
    tok_tab = token_tables.reshape(NUM_TOKEN_FIELDS * VOCAB, D)
    gvec = jnp.full((16,), bn_gamma, jnp.float32)
    bvec = jnp.full((16,), bn_beta, jnp.float32)
    out_lin = _sc_call(tok_tab, seq_table.astype(jnp.float32), tok_idx, s_idx,
                       float_feat.astype(jnp.float32),
                       dense_w.astype(jnp.float32), gvec, bvec)
    out3 = _retile_call(out_lin.reshape(-1))
    return jnp.transpose(out3, (2, 0, 1))


# speedup vs baseline: 1.1116x; 1.0002x over previous
import functools

import jax
import jax.numpy as jnp
from jax import lax
from jax.experimental import pallas as pl
from jax.experimental.pallas import tpu as pltpu
from jax.experimental.pallas import tpu_sc as plsc

NUM_TOKEN_FIELDS = 26
VOCAB = 100000
D = 16
BATCH = 4096
HIST = 50
NUM_FIELDS = NUM_TOKEN_FIELDS + 2

NC = 2
NS = 16
NW = NC * NS
BW = BATCH // NW
SEQ_CHUNK = 64
N_CHUNKS = BW // SEQ_CHUNK
SEQ_STREAMS = SEQ_CHUNK * HIST // 128


def _sc_body(tok_tab, seq_tab, tok_idx, seq_idx, ff, dw, gvec, bvec,
             out,
             tidx_raw, tidx_v, tok_v, sidx_v, seq_rows,
             pooled_v, cntinv_v, ff_v, dense_v, dw_v, g_v, b_v,
             sem_g, sem_w):
    wid = lax.axis_index("s") * NC + lax.axis_index("c")
    base = wid * BW
    iota = lax.iota(jnp.int32, 16)
    zeros16 = jnp.zeros((16,), jnp.float32)

    pltpu.sync_copy(tok_idx.at[pl.ds(base * NUM_TOKEN_FIELDS,
                                     BW * NUM_TOKEN_FIELDS)], tidx_raw)
    pltpu.sync_copy(seq_idx.at[pl.ds(base * HIST, BW * HIST)], sidx_v)
    pltpu.sync_copy(ff, ff_v)
    pltpu.sync_copy(dw, dw_v)
    pltpu.sync_copy(gvec, g_v)
    pltpu.sync_copy(bvec, b_v)

    def tok_xform(f, carry):
        for c in range(BW // 16):
            pos = (iota + c * 16) * NUM_TOKEN_FIELDS + f
            v = plsc.load_gather(tidx_raw, [pos])
            tidx_v[pl.ds(f * BW + c * 16, 16)] = v + f * VOCAB
        return carry

    lax.fori_loop(0, NUM_TOKEN_FIELDS, tok_xform, 0)

    tok_descs = [
        pltpu.async_copy(tok_tab.at[tidx_v.at[pl.ds(f * BW, BW)]],
                         tok_v.at[pl.ds(f * BW, BW)], sem_g)
        for f in range(NUM_TOKEN_FIELDS)
    ]

    def cnt_group(g, carry):
        rows = (iota + g * 16) * HIST
        acc0 = zeros16
        acc1 = zeros16
        for l in range(0, HIST, 2):
            v0 = plsc.load_gather(sidx_v, [rows + l])
            v1 = plsc.load_gather(sidx_v, [rows + l + 1])
            acc0 = acc0 + jnp.where(v0 > 0, 1.0, 0.0).astype(jnp.float32)
            acc1 = acc1 + jnp.where(v1 > 0, 1.0, 0.0).astype(jnp.float32)
        cntinv_v[pl.ds(g * 16, 16)] = 1.0 / jnp.maximum(acc0 + acc1, 1.0)
        return carry

    lax.fori_loop(0, BW // 16, cnt_group, 0)

    def stat_body(i, carry):
        s, s2 = carry
        for u in range(4):
            v = ff_v[pl.ds(i * 64 + u * 16, 16)]
            s = s + v
            s2 = s2 + v * v
        return (s, s2)

    s, s2 = lax.fori_loop(0, BATCH // 64, stat_body, (zeros16, zeros16))
    mean = jnp.sum(s) * (1.0 / BATCH)
    var = jnp.sum(s2) * (1.0 / BATCH) - mean * mean
    x = jnp.full((16,), var + 1e-5, jnp.float32)
    y = plsc.bitcast(jnp.int32(0x5F3759DF)
                     - lax.shift_right_arithmetic(plsc.bitcast(x, jnp.int32), 1),
                     jnp.float32)
    for _ in range(3):
        y = y * (1.5 - 0.5 * x * y * y)
    gamma = g_v[...]
    beta = b_v[...]
    scale = y * gamma
    mean_v = jnp.full((16,), mean, jnp.float32)
    dw_vec = dw_v[...]

    def dense_body(b, carry):
        xv = plsc.load_gather(ff_v, [jnp.full((16,), base + b, jnp.int32)])
        emb = ((xv - mean_v) * scale + beta) * dw_vec
        dense_v[b] = emb
        return carry

    lax.fori_loop(0, BW, dense_body, 0)

    for d_ in tok_descs:
        d_.wait()
    w_descs = [
        pltpu.async_copy(tok_v.at[pl.ds(f * BW, BW)],
                         out.at[pl.ds(base, BW), f], sem_w)
        for f in range(NUM_TOKEN_FIELDS)
    ]

    for cc in range(N_CHUNKS):
        s_descs = [
            pltpu.async_copy(
                seq_tab.at[sidx_v.at[pl.ds(cc * SEQ_CHUNK * HIST + j * 128,
                                           128)]],
                seq_rows.at[pl.ds(j * 128, 128)], sem_g)
            for j in range(SEQ_STREAMS)
        ]
        for d_ in s_descs:
            d_.wait()

        def pool_body(bb, carry):
            r0 = bb * HIST
            a0 = zeros16
            a1 = zeros16
            for l in range(0, HIST, 2):
                a0 = a0 + seq_rows[r0 + l]
                a1 = a1 + seq_rows[r0 + l + 1]
            b = cc * SEQ_CHUNK + bb
            inv = plsc.load_gather(cntinv_v, [jnp.full((16,), b, jnp.int32)])
            pooled_v[b] = (a0 + a1) * inv
            return carry

        lax.fori_loop(0, SEQ_CHUNK, pool_body, 0)

    pw = pltpu.async_copy(pooled_v, out.at[pl.ds(base, BW), NUM_TOKEN_FIELDS],
                          sem_w)
    dwr = pltpu.async_copy(dense_v,
                           out.at[pl.ds(base, BW), NUM_TOKEN_FIELDS + 1],
                           sem_w)
    for d_ in w_descs:
        d_.wait()
    pw.wait()
    dwr.wait()


def _build_call(interpret=False):
    return functools.partial(
        pl.kernel,
        out_type=jax.ShapeDtypeStruct((BATCH, NUM_FIELDS, D), jnp.float32),
        mesh=plsc.VectorSubcoreMesh(core_axis_name="c", subcore_axis_name="s",
                                    num_cores=NC, num_subcores=NS),
        compiler_params=pltpu.CompilerParams(needs_layout_passes=False,
                                             use_tc_tiling_on_sc=False),
        interpret=interpret,
        scratch_types=[
        pltpu.VMEM((BW * NUM_TOKEN_FIELDS,), jnp.int32),
        pltpu.VMEM((NUM_TOKEN_FIELDS * BW,), jnp.int32),
        pltpu.VMEM((NUM_TOKEN_FIELDS * BW, D), jnp.float32),
        pltpu.VMEM((BW * HIST,), jnp.int32),
        pltpu.VMEM((SEQ_CHUNK * HIST, D), jnp.float32),
        pltpu.VMEM((BW, D), jnp.float32),
        pltpu.VMEM((BW,), jnp.float32),
        pltpu.VMEM((BATCH,), jnp.float32),
        pltpu.VMEM((BW, D), jnp.float32),
        pltpu.VMEM((D,), jnp.float32),
        pltpu.VMEM((16,), jnp.float32),
        pltpu.VMEM((16,), jnp.float32),
        pltpu.SemaphoreType.DMA,
        pltpu.SemaphoreType.DMA,
        ],
    )(_sc_body)


_sc_call = _build_call()


def _retile_body(src, out, buf_v, tile_v, sem):
    wid = lax.axis_index("s") * NC + lax.axis_index("c")
    base = wid * BW
    iota = lax.iota(jnp.int32, 16)
    pltpu.sync_copy(src.at[pl.ds(base * NUM_FIELDS * D, BW * NUM_FIELDS * D)],
                    buf_v)

    row = iota * (NUM_FIELDS * D)

    def fld_body(fld, carry):
        def d_body(dd, carry2):
            p0 = row + (fld * D + dd)
            for c in range(BW // 16):
                v = plsc.load_gather(buf_v, [p0 + c * 16 * (NUM_FIELDS * D)])
                tile_v[dd, pl.ds(c * 16, 16)] = v
            return carry2
        lax.fori_loop(0, D, d_body, 0)
        pltpu.sync_copy(tile_v, out.at[fld, pl.ds(0, D), pl.ds(base, BW)])
        return carry

    lax.fori_loop(0, NUM_FIELDS, fld_body, 0)


_retile_call = functools.partial(
    pl.kernel,
    out_type=jax.ShapeDtypeStruct((NUM_FIELDS, D, BATCH), jnp.float32),
    mesh=plsc.VectorSubcoreMesh(core_axis_name="c", subcore_axis_name="s",
                                num_cores=NC, num_subcores=NS),
    compiler_params=pltpu.CompilerParams(needs_layout_passes=False,
                                         use_tc_tiling_on_sc=True),
    scratch_types=[
        pltpu.VMEM((BW * NUM_FIELDS * D,), jnp.float32),
        pltpu.VMEM((D, BW), jnp.float32),
        pltpu.SemaphoreType.DMA,
    ],
)(_retile_body)


def kernel(token_idx, seq_idx, float_feat, token_tables, seq_table,
           dense_w, bn_gamma, bn_beta):
    tok_idx = token_idx.astype(jnp.int32).reshape(-1)
    s_idx = seq_idx.astype(jnp.int32).reshape(-1)
    tok_tab = token_tables.reshape(NUM_TOKEN_FIELDS * VOCAB, D)
    gvec = jnp.full((16,), bn_gamma, jnp.float32)
    bvec = jnp.full((16,), bn_beta, jnp.float32)
    out_lin = _sc_call(tok_tab, seq_table.astype(jnp.float32), tok_idx, s_idx,
                       float_feat.astype(jnp.float32),
                       dense_w.astype(jnp.float32), gvec, bvec)
    out3 = _retile_call(out_lin.reshape(-1))
    return jnp.transpose(out3, (2, 0, 1))
